# Initial kernel scaffold; baseline (speedup 1.0000x reference)
import jax
import jax.numpy as jnp
from jax.experimental import pallas as pl


def kernel(xyz, norm, fps_idx, knn_idx, W1, b1, W2, b2):
    raise NotImplementedError("write your pallas kernel here")


# speedup vs baseline: 3.6638x; 3.6638x over previous
import functools

import jax
import jax.numpy as jnp
from jax import lax
from jax.experimental import pallas as pl
from jax.experimental.pallas import tpu as pltpu

_EPS = 1e-7


def _roll(p, a):
    a %= p.shape[0]
    if a == 0:
        return p
    return jnp.concatenate([p[-a:], p[:-a]], axis=0)


def _dot3(a, b):
    return a[0] * b[0] + a[1] * b[1] + a[2] * b[2]


def _bc(x):
    return x.astype(jnp.bfloat16).astype(jnp.float32)


def _dot3_mxu(a, b):
    return (_bc(a[0]) * _bc(b[0]) + _bc(a[1]) * _bc(b[1])
            + _bc(a[2]) * _bc(b[2]))


def _cross3(a, b):
    return (a[1] * b[2] - a[2] * b[1],
            a[2] * b[0] - a[0] * b[2],
            a[0] * b[1] - a[1] * b[0])


def _risp_body(shift, g_ref, c_ref, w1_ref, b1_ref, w2_ref, b2_ref, out_ref):
    K = g_ref.shape[2]
    S = g_ref.shape[3]

    g = g_ref[0]
    c = c_ref[0]
    gx, gy, gz = g[0], g[1], g[2]
    gnx, gny, gnz = g[3], g[4], g[5]
    cx, cy, cz = c[0:1], c[1:2], c[2:3]
    ncx, ncy, ncz = c[3:4], c[4:5], c[5:6]

    lx, ly, lz = gx - cx, gy - cy, gz - cz
    dp = _dot3_mxu((lx, ly, lz), (ncx, ncy, ncz))
    px, py, pz = lx - dp * ncx, ly - dp * ncy, lz - dp * ncz
    plen = jnp.sqrt(px * px + py * py + pz * pz)
    zero = jnp.zeros_like(plen)
    ux = jnp.where(plen > 0, px / plen, zero)
    uy = jnp.where(plen > 0, py / plen, zero)
    uz = jnp.where(plen > 0, pz / plen, zero)

    ki = lax.broadcasted_iota(jnp.int32, (K, S), 0)
    m = jnp.max(plen, axis=0, keepdims=True)
    lmax = jnp.min(jnp.where(plen == m, ki, K), axis=0, keepdims=True)
    sel = ki == lmax
    vrx = jnp.sum(jnp.where(sel, ux, zero), axis=0, keepdims=True)
    vry = jnp.sum(jnp.where(sel, uy, zero), axis=0, keepdims=True)
    vrz = jnp.sum(jnp.where(sel, uz, zero), axis=0, keepdims=True)

    dots = _dot3_mxu((ux, uy, uz), (vrx, vry, vrz))
    crx, cry, crz = _cross3((ux, uy, uz), (vrx, vry, vrz))
    sv = _dot3_mxu((crx, cry, crz), (ncx, ncy, ncz))
    sg = jnp.sign(sv)
    sg = jnp.where(ki == 0, 1.0, sg)
    d = sg * dots - (1.0 - sg)

    r = jnp.zeros((K, S), jnp.int32)
    for j in range(K):
        dj = d[j:j + 1]
        r = r + (dj > d).astype(jnp.int32)
        r = r + jnp.logical_and(dj == d, ki > j).astype(jnp.int32)

    planes = (lx, ly, lz, gnx, gny, gnz)
    srt = [jnp.zeros((K, S), jnp.float32) for _ in planes]
    rr = r
    rolled = list(planes)
    for a in range(K):
        if a:
            rr = _roll(rr, 1)
            rolled = [_roll(p, 1) for p in rolled]
        mask = rr == ki
        for t in range(len(planes)):
            srt[t] = srt[t] + jnp.where(mask, rolled[t], zero)
    sx, sy, sz, snx, sny, snz = srt

    xi = (sx, sy, sz)
    xin = (snx, sny, snz)
    pn = (ncx, ncy, ncz)
    x3 = tuple(_roll(p, shift) for p in xi)
    x3n = tuple(_roll(p, shift) for p in xin)
    x4 = tuple(_roll(p, -shift) for p in xi)
    x4n = tuple(_roll(p, -shift) for p in xin)

    l0 = jnp.sqrt(_dot3(xi, xi))
    l1 = _roll(l0, shift)
    u0 = tuple(-p / (l0 + _EPS) for p in xi)
    u1 = tuple(-p / (l1 + _EPS) for p in x3)
    offx = tuple(a - b for a, b in zip(xi, x3))
    l2 = jnp.sqrt(_dot3(offx, offx))
    u2 = tuple(p / (l2 + _EPS) for p in offx)

    l4 = _roll(l0, -shift)
    pxi_u = tuple(p / (l4 + _EPS) for p in x4)
    px2_u = tuple(-p for p in u0)
    x2xi_u = tuple(_roll(p, -shift) for p in u2)
    px3_u = tuple(-p for p in u1)
    sn1 = _cross3(pxi_u, px2_u)
    sn2 = _cross3(px3_u, px2_u)

    F = [
        l0,
        -_dot3(u0, pn),
        _dot3(u0, xin),
        _dot3(u0, u1),
        -_dot3(u1, pn),
        _dot3(u1, x3n),
        _dot3(u1, u2),
        -_dot3(u2, xin),
        _dot3(u2, x3n),
        _dot3(pxi_u, px2_u),
        _dot3(pxi_u, x2xi_u),
        _dot3(x2xi_u, x4n),
        _dot3(px2_u, x4n),
        _dot3(sn1, sn2),
    ]

    h1 = []
    for j in range(32):
        acc = F[0] * w1_ref[0, j]
        for cch in range(1, 14):
            acc = acc + F[cch] * w1_ref[cch, j]
        h1.append(jnp.maximum(acc + b1_ref[j], 0.0))

    rows = []
    for j2 in range(64):
        acc = h1[0] * w2_ref[0, j2]
        for j in range(1, 32):
            acc = acc + h1[j] * w2_ref[j, j2]
        mx = jnp.max(acc, axis=0, keepdims=True)
        rows.append(jnp.maximum(mx + b2_ref[j2], 0.0))
    out_ref[0] = jnp.concatenate(rows, axis=0)


def _risp_call(G, C, W1, b1, W2, b2, shift, s_blk):
    B, CH, K, S = G.shape
    grid = (B, S // s_blk)
    body = functools.partial(_risp_body, shift)
    return pl.pallas_call(
        body,
        grid=grid,
        in_specs=[
            pl.BlockSpec((1, CH, K, s_blk), lambda b, j: (b, 0, 0, j)),
            pl.BlockSpec((1, CH, s_blk), lambda b, j: (b, 0, j)),
            pl.BlockSpec(memory_space=pltpu.SMEM),
            pl.BlockSpec(memory_space=pltpu.SMEM),
            pl.BlockSpec(memory_space=pltpu.SMEM),
            pl.BlockSpec(memory_space=pltpu.SMEM),
        ],
        out_specs=pl.BlockSpec((1, 64, s_blk), lambda b, j: (b, 0, j)),
        out_shape=jax.ShapeDtypeStruct((B, 64, S), jnp.float32),
    )(G, C, W1, b1, W2, b2)


def kernel(xyz, norm, fps_idx, knn_idx, W1, b1, W2, b2):
    B, N, _ = xyz.shape
    S = fps_idx.shape[1]
    K = knn_idx.shape[2]

    table = jnp.concatenate(
        [xyz, norm, jnp.zeros((B, N, 2), jnp.float32)], axis=-1)
    ctr = jax.vmap(lambda t, i: t[i])(table, fps_idx)
    grp = jax.vmap(lambda t, i: t[i])(table, knn_idx)

    G = jnp.transpose(grp, (0, 3, 2, 1))
    C = jnp.transpose(ctr, (0, 2, 1))
    shift = 2 if S >= 1024 else 1
    s_blk = 512 if S % 512 == 0 else S
    out = _risp_call(G, C, W1, b1, W2, b2, shift, s_blk)
    new_points = jnp.transpose(out, (0, 2, 1))
    return ctr[..., :3], ctr[..., 3:6], new_points


# speedup vs baseline: 34.5939x; 9.4421x over previous
import functools

import jax
import jax.numpy as jnp
from jax import lax
from jax.experimental import pallas as pl
from jax.experimental.pallas import tpu as pltpu
from jax.experimental.pallas import tpu_sc as plsc

_EPS = 1e-7


def _roll(p, a):
    a %= p.shape[0]
    if a == 0:
        return p
    return jnp.concatenate([p[-a:], p[:-a]], axis=0)


def _dot3(a, b):
    return a[0] * b[0] + a[1] * b[1] + a[2] * b[2]


def _bc(x):
    return x.astype(jnp.bfloat16).astype(jnp.float32)


def _dot3_mxu(a, b):
    return (_bc(a[0]) * _bc(b[0]) + _bc(a[1]) * _bc(b[1])
            + _bc(a[2]) * _bc(b[2]))


def _cross3(a, b):
    return (a[1] * b[2] - a[2] * b[1],
            a[2] * b[0] - a[0] * b[2],
            a[0] * b[1] - a[1] * b[0])


def _risp_body(shift, g_ref, c_ref, w1_ref, b1_ref, w2_ref, b2_ref, out_ref):
    K = g_ref.shape[2]
    S = g_ref.shape[3]

    g = g_ref[0]
    c = c_ref[0]
    gx, gy, gz = g[0], g[1], g[2]
    gnx, gny, gnz = g[3], g[4], g[5]
    cx, cy, cz = c[0:1], c[1:2], c[2:3]
    ncx, ncy, ncz = c[3:4], c[4:5], c[5:6]

    lx, ly, lz = gx - cx, gy - cy, gz - cz
    dp = _dot3_mxu((lx, ly, lz), (ncx, ncy, ncz))
    px, py, pz = lx - dp * ncx, ly - dp * ncy, lz - dp * ncz
    plen = jnp.sqrt(px * px + py * py + pz * pz)
    zero = jnp.zeros_like(plen)
    ux = jnp.where(plen > 0, px / plen, zero)
    uy = jnp.where(plen > 0, py / plen, zero)
    uz = jnp.where(plen > 0, pz / plen, zero)

    ki = lax.broadcasted_iota(jnp.int32, (K, S), 0)
    m = jnp.max(plen, axis=0, keepdims=True)
    lmax = jnp.min(jnp.where(plen == m, ki, K), axis=0, keepdims=True)
    sel = ki == lmax
    vrx = jnp.sum(jnp.where(sel, ux, zero), axis=0, keepdims=True)
    vry = jnp.sum(jnp.where(sel, uy, zero), axis=0, keepdims=True)
    vrz = jnp.sum(jnp.where(sel, uz, zero), axis=0, keepdims=True)

    dots = _dot3_mxu((ux, uy, uz), (vrx, vry, vrz))
    crx, cry, crz = _cross3((ux, uy, uz), (vrx, vry, vrz))
    sv = _dot3_mxu((crx, cry, crz), (ncx, ncy, ncz))
    sg = jnp.sign(sv)
    sg = jnp.where(ki == 0, 1.0, sg)
    d = sg * dots - (1.0 - sg)

    r = jnp.zeros((K, S), jnp.int32)
    for j in range(K):
        dj = d[j:j + 1]
        r = r + (dj > d).astype(jnp.int32)
        r = r + jnp.logical_and(dj == d, ki > j).astype(jnp.int32)

    planes = (lx, ly, lz, gnx, gny, gnz)
    srt = [jnp.zeros((K, S), jnp.float32) for _ in planes]
    rr = r
    rolled = list(planes)
    for a in range(K):
        if a:
            rr = _roll(rr, 1)
            rolled = [_roll(p, 1) for p in rolled]
        mask = rr == ki
        for t in range(len(planes)):
            srt[t] = srt[t] + jnp.where(mask, rolled[t], zero)
    sx, sy, sz, snx, sny, snz = srt

    xi = (sx, sy, sz)
    xin = (snx, sny, snz)
    pn = (ncx, ncy, ncz)
    x3n = tuple(_roll(p, shift) for p in xin)
    x3 = tuple(_roll(p, shift) for p in xi)
    x4 = tuple(_roll(p, -shift) for p in xi)
    x4n = tuple(_roll(p, -shift) for p in xin)

    l0 = jnp.sqrt(_dot3(xi, xi))
    l1 = _roll(l0, shift)
    u0 = tuple(-p / (l0 + _EPS) for p in xi)
    u1 = tuple(-p / (l1 + _EPS) for p in x3)
    offx = tuple(a - b for a, b in zip(xi, x3))
    l2 = jnp.sqrt(_dot3(offx, offx))
    u2 = tuple(p / (l2 + _EPS) for p in offx)

    l4 = _roll(l0, -shift)
    pxi_u = tuple(p / (l4 + _EPS) for p in x4)
    px2_u = tuple(-p for p in u0)
    x2xi_u = tuple(_roll(p, -shift) for p in u2)
    px3_u = tuple(-p for p in u1)
    sn1 = _cross3(pxi_u, px2_u)
    sn2 = _cross3(px3_u, px2_u)

    F = [
        l0,
        -_dot3(u0, pn),
        _dot3(u0, xin),
        _dot3(u0, u1),
        -_dot3(u1, pn),
        _dot3(u1, x3n),
        _dot3(u1, u2),
        -_dot3(u2, xin),
        _dot3(u2, x3n),
        _dot3(pxi_u, px2_u),
        _dot3(pxi_u, x2xi_u),
        _dot3(x2xi_u, x4n),
        _dot3(px2_u, x4n),
        _dot3(sn1, sn2),
    ]

    h1 = []
    for j in range(32):
        acc = F[0] * w1_ref[0, j]
        for cch in range(1, 14):
            acc = acc + F[cch] * w1_ref[cch, j]
        h1.append(jnp.maximum(acc + b1_ref[j], 0.0))

    rows = []
    for j2 in range(64):
        acc = h1[0] * w2_ref[0, j2]
        for j in range(1, 32):
            acc = acc + h1[j] * w2_ref[j, j2]
        mx = jnp.max(acc, axis=0, keepdims=True)
        rows.append(jnp.maximum(mx + b2_ref[j2], 0.0))
    out_ref[0] = jnp.concatenate(rows, axis=0)


def _risp_call(G, C, W1, b1, W2, b2, shift, s_blk):
    B, CH, K, S = G.shape
    grid = (B, S // s_blk)
    body = functools.partial(_risp_body, shift)
    return pl.pallas_call(
        body,
        grid=grid,
        in_specs=[
            pl.BlockSpec((1, CH, K, s_blk), lambda b, j: (b, 0, 0, j)),
            pl.BlockSpec((1, CH, s_blk), lambda b, j: (b, 0, j)),
            pl.BlockSpec(memory_space=pltpu.SMEM),
            pl.BlockSpec(memory_space=pltpu.SMEM),
            pl.BlockSpec(memory_space=pltpu.SMEM),
            pl.BlockSpec(memory_space=pltpu.SMEM),
        ],
        out_specs=pl.BlockSpec((1, 64, s_blk), lambda b, j: (b, 0, j)),
        out_shape=jax.ShapeDtypeStruct((B, 64, S), jnp.float32),
    )(G, C, W1, b1, W2, b2)


_NW = 32
_CHUNK = 128


def _sc_gather(table, idx):
    TOT = idx.shape[0]
    CH = table.shape[1]
    per_w = TOT // _NW
    half_rows = per_w // 2
    half_chunks = half_rows // _CHUNK
    mesh = plsc.VectorSubcoreMesh(core_axis_name="c", subcore_axis_name="s")

    @functools.partial(
        pl.kernel, mesh=mesh,
        out_type=jax.ShapeDtypeStruct((TOT, CH), jnp.float32),
        compiler_params=pltpu.CompilerParams(use_tc_tiling_on_sc=False),
        scratch_types=[
            pltpu.VMEM((per_w,), jnp.int32),
            pltpu.VMEM((half_rows, CH), jnp.float32),
            pltpu.SemaphoreType.DMA,
        ],
    )
    def gk(table_hbm, idx_hbm, out_hbm, idx_v, rows_v, sem):
        wid = lax.axis_index("s") * 2 + lax.axis_index("c")
        base = wid * per_w
        pltpu.sync_copy(idx_hbm.at[pl.ds(base, per_w)], idx_v)
        for h in range(2):
            def fire(i, carry):
                pltpu.async_copy(
                    table_hbm.at[idx_v.at[pl.ds(h * half_rows + i * _CHUNK,
                                                _CHUNK)]],
                    rows_v.at[pl.ds(i * _CHUNK, _CHUNK)],
                    sem)
                return carry

            def drain(i, carry):
                pltpu.make_async_copy(
                    table_hbm.at[idx_v.at[pl.ds(h * half_rows + i * _CHUNK,
                                                _CHUNK)]],
                    rows_v.at[pl.ds(i * _CHUNK, _CHUNK)],
                    sem).wait()
                return carry

            lax.fori_loop(0, half_chunks, fire, 0)
            lax.fori_loop(0, half_chunks, drain, 0)
            pltpu.sync_copy(rows_v,
                            out_hbm.at[pl.ds(base + h * half_rows, half_rows)])

    return gk(table, idx)


def kernel(xyz, norm, fps_idx, knn_idx, W1, b1, W2, b2):
    B, N, _ = xyz.shape
    S = fps_idx.shape[1]
    K = knn_idx.shape[2]

    table = jnp.concatenate(
        [xyz, norm, jnp.zeros((B, N, 2), jnp.float32)], axis=-1)
    offs = (jnp.arange(B, dtype=jnp.int32) * N)[:, None]
    idx_all = jnp.concatenate([
        (knn_idx.reshape(B, S * K).astype(jnp.int32) + offs).reshape(-1),
        (fps_idx.astype(jnp.int32) + offs).reshape(-1),
    ])
    tot = B * S * K + B * S
    pad = (-tot) % (_NW * _CHUNK * 2)
    if pad:
        idx_all = jnp.concatenate(
            [idx_all, jnp.zeros((pad,), jnp.int32)])
    rows = _sc_gather(table.reshape(B * N, 8), idx_all)
    grp = rows[:B * S * K].reshape(B, S, K, 8)
    ctr = rows[B * S * K:tot].reshape(B, S, 8)

    G = jnp.transpose(grp, (0, 3, 2, 1))
    C = jnp.transpose(ctr, (0, 2, 1))
    shift = 2 if S >= 1024 else 1
    s_blk = 512 if S % 512 == 0 else S
    out = _risp_call(G, C, W1, b1, W2, b2, shift, s_blk)
    new_points = jnp.transpose(out, (0, 2, 1))
    return ctr[..., :3], ctr[..., 3:6], new_points
